# Initial kernel scaffold; baseline (speedup 1.0000x reference)
import jax
import jax.numpy as jnp
from jax.experimental import pallas as pl


def kernel(query, keys, values):
    raise NotImplementedError("write your pallas kernel here")


# speedup vs baseline: 1.6290x; 1.6290x over previous
import functools

import jax
import jax.numpy as jnp
from jax.experimental import pallas as pl
from jax.experimental.pallas import tpu as pltpu

_MEM = 1_000_000
_D = 64
_K = 50
_RB = 8000
_NBLK = _MEM // _RB
_PAD = 1_048_576
_ROWS = 8192
_LANES = 128
_CHUNK = 64
_NCHUNK = _ROWS // _CHUNK


def _dist_kernel(q_ref, k_ref, o_ref):
    q = q_ref[...]
    k = k_ref[...]
    d = k - q
    o_ref[...] = jnp.sum(d * d, axis=1).reshape(1, 1, _RB)


def _select_kernel(d_ref, v_ref, o_ref, s_ref):
    s_ref[...] = d_ref[...]

    row_i = jax.lax.broadcasted_iota(jnp.int32, (_CHUNK, _LANES), 0)
    lane_i = jax.lax.broadcasted_iota(jnp.int32, (_CHUNK, _LANES), 1)
    flat_i = jax.lax.broadcasted_iota(jnp.int32, (_NCHUNK, _LANES), 0) * _LANES \
        + jax.lax.broadcasted_iota(jnp.int32, (_NCHUNK, _LANES), 1)
    big = jnp.int32(2**30)

    m0 = jnp.min(s_ref[...].reshape(_NCHUNK, _CHUNK, _LANES), axis=1)

    def body(_, carry):
        m, num, den = carry
        mval = jnp.min(m)
        flat = jnp.min(jnp.where(m == mval, flat_i, big))
        c = flat // _LANES
        l = flat % _LANES
        slab = s_ref[pl.ds(c * _CHUNK, _CHUNK), :]
        vslab = v_ref[pl.ds(c * _CHUNK, _CHUNK), :]
        cand = (slab == mval) & (lane_i == l)
        rid = jnp.min(jnp.where(cand, row_i, big))
        emask = cand & (row_i == rid)
        val = jnp.sum(jnp.where(emask, vslab, 0.0))
        w = 1.0 / (mval + 1e-7)
        slab = jnp.where(emask, jnp.inf, slab)
        s_ref[pl.ds(c * _CHUNK, _CHUNK), :] = slab
        newrow = jnp.min(slab, axis=0)
        crow = jax.lax.broadcasted_iota(jnp.int32, (_NCHUNK, _LANES), 0)
        m = jnp.where(crow == c, newrow[None, :], m)
        return m, num + w * val, den + w

    _, num, den = jax.lax.fori_loop(
        0, _K, body, (m0, jnp.float32(0.0), jnp.float32(0.0)))
    o_ref[0, 0] = num / den


@jax.jit
def kernel(query, keys, values):
    dist = pl.pallas_call(
        _dist_kernel,
        grid=(_NBLK,),
        in_specs=[
            pl.BlockSpec((1, _D), lambda b: (0, 0)),
            pl.BlockSpec((_RB, _D), lambda b: (b, 0)),
        ],
        out_specs=pl.BlockSpec((1, 1, _RB), lambda b: (b, 0, 0)),
        out_shape=jax.ShapeDtypeStruct((_NBLK, 1, _RB), jnp.float32),
    )(query, keys)

    dist_flat = dist.reshape(_MEM)
    pad = _PAD - _MEM
    dist_pad = jnp.concatenate(
        [dist_flat, jnp.full((pad,), jnp.inf, jnp.float32)]
    ).reshape(_ROWS, _LANES)
    vals_pad = jnp.concatenate(
        [values, jnp.zeros((pad,), jnp.float32)]
    ).reshape(_ROWS, _LANES)

    out = pl.pallas_call(
        _select_kernel,
        out_shape=jax.ShapeDtypeStruct((1, 1), jnp.float32),
        out_specs=pl.BlockSpec(memory_space=pltpu.SMEM),
        scratch_shapes=[pltpu.VMEM((_ROWS, _LANES), jnp.float32)],
    )(dist_pad, vals_pad)
    return out[0, 0]


# speedup vs baseline: 2.3171x; 1.4224x over previous
import functools

import jax
import jax.numpy as jnp
from jax.experimental import pallas as pl
from jax.experimental.pallas import tpu as pltpu

_MEM = 1_000_000
_D = 64
_K = 50
_RB = 20000
_NBLK = _MEM // _RB
_PAD = 1_048_576
_ROWS = 8192
_LANES = 128
_CHUNK = 64
_NCHUNK = _ROWS // _CHUNK


def _dist_kernel(q_ref, k_ref, o_ref):
    d = k_ref[...] - q_ref[...]
    s = d * d
    s_hi = s.astype(jnp.bfloat16)
    s_lo = (s - s_hi.astype(jnp.float32)).astype(jnp.bfloat16)
    ones = jnp.ones((1, _D), jnp.bfloat16)
    dn = (((1,), (1,)), ((), ()))
    row = (
        jax.lax.dot_general(ones, s_hi, dimension_numbers=dn,
                            preferred_element_type=jnp.float32)
        + jax.lax.dot_general(ones, s_lo, dimension_numbers=dn,
                              preferred_element_type=jnp.float32)
    )
    o_ref[...] = row.reshape(1, 1, _RB)


def _select_kernel(d_ref, v_ref, o_ref, s_ref):
    s_ref[...] = d_ref[...]

    row_i = jax.lax.broadcasted_iota(jnp.int32, (_CHUNK, _LANES), 0)
    lane_i = jax.lax.broadcasted_iota(jnp.int32, (_CHUNK, _LANES), 1)
    flat_i = jax.lax.broadcasted_iota(jnp.int32, (_NCHUNK, _LANES), 0) * _LANES \
        + jax.lax.broadcasted_iota(jnp.int32, (_NCHUNK, _LANES), 1)
    big = jnp.int32(2**30)

    m0 = jnp.min(s_ref[...].reshape(_NCHUNK, _CHUNK, _LANES), axis=1)

    def body(_, carry):
        m, num, den = carry
        mval = jnp.min(m)
        flat = jnp.min(jnp.where(m == mval, flat_i, big))
        c = flat // _LANES
        l = flat % _LANES
        slab = s_ref[pl.ds(c * _CHUNK, _CHUNK), :]
        vslab = v_ref[pl.ds(c * _CHUNK, _CHUNK), :]
        cand = (slab == mval) & (lane_i == l)
        rid = jnp.min(jnp.where(cand, row_i, big))
        emask = cand & (row_i == rid)
        val = jnp.sum(jnp.where(emask, vslab, 0.0))
        w = 1.0 / (mval + 1e-7)
        slab = jnp.where(emask, jnp.inf, slab)
        s_ref[pl.ds(c * _CHUNK, _CHUNK), :] = slab
        newrow = jnp.min(slab, axis=0)
        crow = jax.lax.broadcasted_iota(jnp.int32, (_NCHUNK, _LANES), 0)
        m = jnp.where(crow == c, newrow[None, :], m)
        return m, num + w * val, den + w

    _, num, den = jax.lax.fori_loop(
        0, _K, body, (m0, jnp.float32(0.0), jnp.float32(0.0)))
    o_ref[0, 0] = num / den


@jax.jit
def kernel(query, keys, values):
    dist = pl.pallas_call(
        _dist_kernel,
        grid=(_NBLK,),
        in_specs=[
            pl.BlockSpec((1, _D), lambda b: (0, 0)),
            pl.BlockSpec((_RB, _D), lambda b: (b, 0)),
        ],
        out_specs=pl.BlockSpec((1, 1, _RB), lambda b: (b, 0, 0)),
        out_shape=jax.ShapeDtypeStruct((_NBLK, 1, _RB), jnp.float32),
        compiler_params=pltpu.CompilerParams(
            dimension_semantics=("parallel",)),
    )(query, keys)

    dist_flat = dist.reshape(_MEM)
    pad = _PAD - _MEM
    dist_pad = jnp.concatenate(
        [dist_flat, jnp.full((pad,), jnp.inf, jnp.float32)]
    ).reshape(_ROWS, _LANES)
    vals_pad = jnp.concatenate(
        [values, jnp.zeros((pad,), jnp.float32)]
    ).reshape(_ROWS, _LANES)

    out = pl.pallas_call(
        _select_kernel,
        out_shape=jax.ShapeDtypeStruct((1, 1), jnp.float32),
        out_specs=pl.BlockSpec(memory_space=pltpu.SMEM),
        scratch_shapes=[pltpu.VMEM((_ROWS, _LANES), jnp.float32)],
    )(dist_pad, vals_pad)
    return out[0, 0]


# speedup vs baseline: 2.3674x; 1.0217x over previous
import functools

import jax
import jax.numpy as jnp
from jax.experimental import pallas as pl
from jax.experimental.pallas import tpu as pltpu

_MEM = 1_000_000
_D = 64
_K = 50
_RB = 20000
_NBLK = _MEM // _RB
_PAD = 1_048_576
_ROWS = 8192
_LANES = 128
_CHUNK = 64
_NCHUNK = _ROWS // _CHUNK


def _dist_kernel(q_ref, k_ref, o_ref):
    d = k_ref[...] - q_ref[...]
    s = d * d
    hi_f = jax.lax.bitcast_convert_type(
        jax.lax.bitcast_convert_type(s, jnp.uint32) & jnp.uint32(0xFFFF0000),
        jnp.float32)
    s_hi = hi_f.astype(jnp.bfloat16)
    s_lo = (s - hi_f).astype(jnp.bfloat16)
    cat = jnp.concatenate([s_hi, s_lo], axis=1)
    ones = jnp.ones((1, 2 * _D), jnp.bfloat16)
    dn = (((1,), (1,)), ((), ()))
    row = jax.lax.dot_general(ones, cat, dimension_numbers=dn,
                              preferred_element_type=jnp.float32)
    o_ref[...] = row.reshape(1, 1, _RB)


def _select_kernel(d_ref, v_ref, o_ref):
    def body(_, carry):
        lo, hi = carry
        mid = lo + (hi - lo) // 2
        t = jax.lax.bitcast_convert_type(mid, jnp.float32)
        cnt = jnp.sum((d_ref[...] <= t).astype(jnp.float32))
        take = cnt >= jnp.float32(_K)
        return jnp.where(take, lo, mid + 1), jnp.where(take, mid, hi)

    inf_bits = jnp.int32(0x7F800000)
    lo, hi = jax.lax.fori_loop(
        0, 31, body, (jnp.int32(0), inf_bits))
    thr = jax.lax.bitcast_convert_type(hi, jnp.float32)

    d = d_ref[...]
    w = jnp.where(d <= thr, 1.0 / (d + 1e-7), 0.0)
    num = jnp.sum(w * v_ref[...])
    den = jnp.sum(w)
    o_ref[0, 0] = num / den


@jax.jit
def kernel(query, keys, values):
    dist = pl.pallas_call(
        _dist_kernel,
        grid=(_NBLK,),
        in_specs=[
            pl.BlockSpec((1, _D), lambda b: (0, 0)),
            pl.BlockSpec((_RB, _D), lambda b: (b, 0)),
        ],
        out_specs=pl.BlockSpec((1, 1, _RB), lambda b: (b, 0, 0)),
        out_shape=jax.ShapeDtypeStruct((_NBLK, 1, _RB), jnp.float32),
        compiler_params=pltpu.CompilerParams(
            dimension_semantics=("parallel",)),
    )(query, keys)

    dist_flat = dist.reshape(_MEM)
    pad = _PAD - _MEM
    dist_pad = jnp.concatenate(
        [dist_flat, jnp.full((pad,), jnp.inf, jnp.float32)]
    ).reshape(_ROWS, _LANES)
    vals_pad = jnp.concatenate(
        [values, jnp.zeros((pad,), jnp.float32)]
    ).reshape(_ROWS, _LANES)

    out = pl.pallas_call(
        _select_kernel,
        out_shape=jax.ShapeDtypeStruct((1, 1), jnp.float32),
        out_specs=pl.BlockSpec(memory_space=pltpu.SMEM),
    )(dist_pad, vals_pad)
    return out[0, 0]


# speedup vs baseline: 2.5800x; 1.0898x over previous
import jax
import jax.numpy as jnp
from jax.experimental import pallas as pl
from jax.experimental.pallas import tpu as pltpu

_MEM = 1_000_000
_D = 64
_K = 50
_RB = 20000
_NBLK = _MEM // _RB
_PAD = 1_048_576
_ROWS = 8192
_LANES = 128
_CHUNK = 64
_NCHUNK = _ROWS // _CHUNK


def _dist_kernel(q_ref, k_ref, o_ref, x_scr):
    h = _RB // 2
    x_scr[:, 0:_D] = k_ref[0:h, :]
    x_scr[:, _D:2 * _D] = k_ref[h:_RB, :]
    q2 = jnp.concatenate([q_ref[...], q_ref[...]], axis=1)
    d = x_scr[...] - q2
    s = d * d
    hi_f = jax.lax.bitcast_convert_type(
        jax.lax.bitcast_convert_type(s, jnp.uint32) & jnp.uint32(0xFFFF0000),
        jnp.float32)
    s_hi = hi_f.astype(jnp.bfloat16)
    s_lo = (s - hi_f).astype(jnp.bfloat16)
    cat = jnp.concatenate([s_hi, s_lo], axis=1)
    lane = jax.lax.broadcasted_iota(jnp.int32, (2, 4 * _D), 1)
    row = jax.lax.broadcasted_iota(jnp.int32, (2, 4 * _D), 0)
    sel = (lane % (2 * _D)) < _D
    ones = jnp.where((row == 0) == sel, 1.0, 0.0).astype(jnp.bfloat16)
    dn = (((1,), (1,)), ((), ()))
    res = jax.lax.dot_general(ones, cat, dimension_numbers=dn,
                              preferred_element_type=jnp.float32)
    o_ref[...] = res.reshape(1, 2, h)


def _select_kernel(d_ref, v_ref, o_ref):
    def body(_, carry):
        lo, hi = carry
        mid = lo + (hi - lo) // 2
        t = jax.lax.bitcast_convert_type(mid, jnp.float32)
        cnt = jnp.sum((d_ref[...] <= t).astype(jnp.float32))
        take = cnt >= jnp.float32(_K)
        return jnp.where(take, lo, mid + 1), jnp.where(take, mid, hi)

    inf_bits = jnp.int32(0x7F800000)
    lo, hi = jax.lax.fori_loop(
        0, 31, body, (jnp.int32(0), inf_bits))
    thr = jax.lax.bitcast_convert_type(hi, jnp.float32)

    d = d_ref[...]
    w = jnp.where(d <= thr, 1.0 / (d + 1e-7), 0.0)
    num = jnp.sum(w * v_ref[...])
    den = jnp.sum(w)
    o_ref[0, 0] = num / den


@jax.jit
def kernel(query, keys, values):
    dist = pl.pallas_call(
        _dist_kernel,
        grid=(_NBLK,),
        in_specs=[
            pl.BlockSpec((1, _D), lambda b: (0, 0)),
            pl.BlockSpec((_RB, _D), lambda b: (b, 0)),
        ],
        out_specs=pl.BlockSpec((1, 2, _RB // 2), lambda b: (b, 0, 0)),
        out_shape=jax.ShapeDtypeStruct((_NBLK, 2, _RB // 2), jnp.float32),
        compiler_params=pltpu.CompilerParams(
            dimension_semantics=("parallel",)),
        scratch_shapes=[pltpu.VMEM((_RB // 2, 2 * _D), jnp.float32)],
    )(query, keys)

    dist_flat = dist.reshape(_MEM)
    pad = _PAD - _MEM
    dist_pad = jnp.concatenate(
        [dist_flat, jnp.full((pad,), jnp.inf, jnp.float32)]
    ).reshape(_ROWS, _LANES)
    vals_pad = jnp.concatenate(
        [values, jnp.zeros((pad,), jnp.float32)]
    ).reshape(_ROWS, _LANES)

    out = pl.pallas_call(
        _select_kernel,
        out_shape=jax.ShapeDtypeStruct((1, 1), jnp.float32),
        out_specs=pl.BlockSpec(memory_space=pltpu.SMEM),
    )(dist_pad, vals_pad)
    return out[0, 0]
